# baseline (speedup 1.0000x reference)
import functools

import jax
import jax.numpy as jnp
from jax import lax
from jax.experimental import pallas as pl
from jax.experimental.pallas import tpu as pltpu
from jax.experimental.pallas import tpu_sc as plsc

LANES = 16
N_TILES = 16
E_CHUNK = 112
NBUF = 3
NEBUF = 6
GLA = 1
ELA = 2


def _make_sc_agg(n_nodes, n_chunks, half):
    rows_main = (n_nodes // (N_TILES * 8)) * 8
    rows_last = n_nodes - (N_TILES - 1) * rows_main
    n_groups = n_chunks // NEBUF
    mesh = plsc.VectorSubcoreMesh(core_axis_name="c", subcore_axis_name="s")

    @functools.partial(
        pl.kernel,
        mesh=mesh,
        out_type=jax.ShapeDtypeStruct((n_nodes, 2 * half), jnp.float32),
        scratch_types=[
            pltpu.VMEM((NEBUF, 3, E_CHUNK), jnp.int32),
            pltpu.VMEM((NBUF, E_CHUNK, half), jnp.float32),
            pltpu.VMEM_SHARED((n_nodes, half), jnp.float32),
            pltpu.SemaphoreType.DMA((NEBUF,)),
            pltpu.SemaphoreType.DMA((NBUF,)),
            pltpu.SemaphoreType.DMA((NBUF,)),
        ],
    )
    def sc_agg(y_lo, y_hi, edata_h, bias_h, out,
               ebuf, rows_v, acc, isem, gsem, ssem):
        c = lax.axis_index("c")
        s = lax.axis_index("s")
        row_base = pl.multiple_of(s * rows_main, 8)
        last_base = (N_TILES - 1) * rows_main

        @pl.when(s < N_TILES - 1)
        def _():
            pltpu.sync_copy(bias_h.at[c, pl.ds(0, rows_main)],
                            acc.at[pl.ds(row_base, rows_main)])

        @pl.when(s == N_TILES - 1)
        def _():
            pltpu.sync_copy(bias_h.at[c],
                            acc.at[pl.ds(last_base, rows_last)])

        plsc.subcore_barrier()

        def edge_pass(yh):
            for j in range(ELA):
                pltpu.async_copy(edata_h.at[s, j], ebuf.at[j], isem.at[j])
            for j in range(GLA):
                pltpu.make_async_copy(edata_h.at[s, j], ebuf.at[j],
                                      isem.at[j]).wait()
                pltpu.async_copy(yh.at[ebuf.at[j, 0]], rows_v.at[j],
                                 gsem.at[j])

            def group_body(g, carry):
                for u in range(NEBUF):
                    k = g * NEBUF + u
                    b = u % NBUF
                    eslot = (u + ELA) % NEBUF

                    @pl.when(k + ELA < n_chunks)
                    def _(eslot=eslot, k=k):
                        pltpu.async_copy(edata_h.at[s, k + ELA],
                                         ebuf.at[eslot], isem.at[eslot])

                    bn = (u + GLA) % NBUF
                    en = (u + GLA) % NEBUF

                    @pl.when(jnp.logical_and(k + GLA < n_chunks,
                                             k + GLA >= NBUF))
                    def _(bn=bn):
                        pltpu.make_async_copy(rows_v.at[bn],
                                              acc.at[ebuf.at[0, 1]],
                                              ssem.at[bn]).wait()

                    @pl.when(k + GLA < n_chunks)
                    def _(bn=bn, en=en):
                        pltpu.make_async_copy(edata_h.at[s, 0], ebuf.at[en],
                                              isem.at[en]).wait()
                        pltpu.async_copy(yh.at[ebuf.at[en, 0]],
                                         rows_v.at[bn], gsem.at[bn])

                    pltpu.make_async_copy(yh.at[ebuf.at[u, 0]], rows_v.at[b],
                                          gsem.at[b]).wait()

                    def mul_body(gi, carry2, b=b, u=u):
                        wi = ebuf[u, 2, pl.ds(gi * LANES, LANES)]
                        ew16 = lax.bitcast_convert_type(wi, jnp.float32)
                        for e in range(LANES):
                            w = lax.gather(
                                ew16,
                                jnp.full((LANES, 1), e, jnp.int32),
                                lax.GatherDimensionNumbers(
                                    offset_dims=(),
                                    collapsed_slice_dims=(0,),
                                    start_index_map=(0,)),
                                slice_sizes=(1,),
                                mode=lax.GatherScatterMode.PROMISE_IN_BOUNDS)
                            i = gi * LANES + e
                            for j in range(half // LANES):
                                sl = pl.ds(j * LANES, LANES)
                                rows_v[b, i, sl] = rows_v[b, i, sl] * w
                        return carry2

                    lax.fori_loop(0, E_CHUNK // LANES, mul_body, 0)
                    pltpu.async_copy(rows_v.at[b], acc.at[ebuf.at[u, 1]],
                                     ssem.at[b], add=True)
                return carry

            lax.fori_loop(0, n_groups, group_body, 0)
            for b in range(NBUF):
                pltpu.make_async_copy(rows_v.at[b], acc.at[ebuf.at[0, 1]],
                                      ssem.at[b]).wait()

        @pl.when(c == 0)
        def _():
            edge_pass(y_lo)

        @pl.when(c == 1)
        def _():
            edge_pass(y_hi)

        plsc.subcore_barrier()

        for core in range(2):
            col = core * half

            @pl.when(jnp.logical_and(c == core, s < N_TILES - 1))
            def _(col=col):
                pltpu.sync_copy(
                    acc.at[pl.ds(row_base, rows_main)],
                    out.at[pl.ds(row_base, rows_main), pl.ds(col, half)])

            @pl.when(jnp.logical_and(c == core, s == N_TILES - 1))
            def _(col=col):
                pltpu.sync_copy(
                    acc.at[pl.ds(last_base, rows_last)],
                    out.at[pl.ds(last_base, rows_last), pl.ds(col, half)])

    return sc_agg


def _mm_body(x_ref, w_ref, o_ref, *, half):
    x = x_ref[...]
    w = w_ref[...]
    o_ref[0] = jnp.dot(x, w[:, :half], preferred_element_type=jnp.float32)
    o_ref[1] = jnp.dot(x, w[:, half:], preferred_element_type=jnp.float32)


def kernel(x, edge_index, edge_weight, W, b):
    n_nodes, d_in = x.shape
    d_out = W.shape[1]
    n_edges = edge_weight.shape[0]
    half = d_out // 2

    cpt = -(-(n_edges // N_TILES) // (E_CHUNK * NEBUF)) * NEBUF
    n_pad = N_TILES * cpt * E_CHUNK - n_edges
    pad_idx = (jnp.arange(n_pad, dtype=jnp.int32) * 8) % n_nodes
    src = jnp.concatenate([edge_index[0].astype(jnp.int32), pad_idx])
    dst = jnp.concatenate([edge_index[1].astype(jnp.int32), pad_idx])
    ew = jnp.concatenate([edge_weight.astype(jnp.float32),
                          jnp.zeros((n_pad,), jnp.float32)])
    edata = jnp.stack(
        [src, dst, lax.bitcast_convert_type(ew, jnp.int32)])
    edata = edata.reshape(3, N_TILES, cpt, E_CHUNK).transpose(1, 2, 0, 3)

    rows_main = (n_nodes // (N_TILES * 8)) * 8
    rows_last = n_nodes - (N_TILES - 1) * rows_main
    bias2 = jnp.broadcast_to(b, (rows_last, d_out))
    bias2 = bias2.reshape(rows_last, 2, half).transpose(1, 0, 2)

    rows_blk = 1000
    grid = (n_nodes // rows_blk,)
    y2 = pl.pallas_call(
        functools.partial(_mm_body, half=half),
        grid=grid,
        in_specs=[
            pl.BlockSpec((rows_blk, d_in), lambda i: (i, 0)),
            pl.BlockSpec((d_in, d_out), lambda i: (0, 0)),
        ],
        out_specs=pl.BlockSpec((2, rows_blk, half), lambda i: (0, i, 0)),
        out_shape=jax.ShapeDtypeStruct((2, n_nodes, half), jnp.float32),
    )(x, W)

    sc_agg = _make_sc_agg(n_nodes, cpt, half)
    return sc_agg(y2[0], y2[1], edata, bias2)


# speedup vs baseline: 1.0200x; 1.0200x over previous
import functools

import jax
import jax.numpy as jnp
from jax import lax
from jax.experimental import pallas as pl
from jax.experimental.pallas import tpu as pltpu
from jax.experimental.pallas import tpu_sc as plsc

LANES = 16
N_TILES = 16
E_CHUNK = 80
NBUF = 4
NEBUF = 8
GLA = 2
ELA = 3


def _make_sc_agg(n_nodes, n_chunks, half):
    rows_main = (n_nodes // (N_TILES * 8)) * 8
    rows_last = n_nodes - (N_TILES - 1) * rows_main
    n_groups = n_chunks // NEBUF
    mesh = plsc.VectorSubcoreMesh(core_axis_name="c", subcore_axis_name="s")

    @functools.partial(
        pl.kernel,
        mesh=mesh,
        out_type=jax.ShapeDtypeStruct((n_nodes, 2 * half), jnp.float32),
        scratch_types=[
            pltpu.VMEM((NEBUF, 3, E_CHUNK), jnp.int32),
            pltpu.VMEM((NBUF, E_CHUNK, half), jnp.float32),
            pltpu.VMEM_SHARED((n_nodes, half), jnp.float32),
            pltpu.SemaphoreType.DMA((NEBUF,)),
            pltpu.SemaphoreType.DMA((NBUF,)),
            pltpu.SemaphoreType.DMA((NBUF,)),
        ],
    )
    def sc_agg(y_lo, y_hi, src3_h, dst3_h, ewi3_h, bias_h, out,
               ebuf, rows_v, acc, isem, gsem, ssem):
        c = lax.axis_index("c")
        s = lax.axis_index("s")
        row_base = pl.multiple_of(s * rows_main, 8)
        last_base = (N_TILES - 1) * rows_main

        @pl.when(s < N_TILES - 1)
        def _():
            pltpu.sync_copy(bias_h.at[c, pl.ds(0, rows_main)],
                            acc.at[pl.ds(row_base, rows_main)])

        @pl.when(s == N_TILES - 1)
        def _():
            pltpu.sync_copy(bias_h.at[c],
                            acc.at[pl.ds(last_base, rows_last)])

        plsc.subcore_barrier()

        def stage_edata(kk, slot):
            pltpu.async_copy(src3_h.at[s, kk], ebuf.at[slot, 0],
                             isem.at[slot])
            pltpu.async_copy(dst3_h.at[s, kk], ebuf.at[slot, 1],
                             isem.at[slot])
            pltpu.async_copy(ewi3_h.at[s, kk], ebuf.at[slot, 2],
                             isem.at[slot])

        def wait_edata(slot):
            for j in range(3):
                pltpu.make_async_copy(src3_h.at[s, 0], ebuf.at[slot, j],
                                      isem.at[slot]).wait()

        def edge_pass(yh):
            for j in range(ELA):
                stage_edata(j, j)
            for j in range(GLA):
                wait_edata(j)
                pltpu.async_copy(yh.at[ebuf.at[j, 0]], rows_v.at[j],
                                 gsem.at[j])

            def group_body(g, carry):
                for u in range(NEBUF):
                    k = g * NEBUF + u
                    b = u % NBUF
                    eslot = (u + ELA) % NEBUF

                    @pl.when(k + ELA < n_chunks)
                    def _(eslot=eslot, k=k):
                        stage_edata(k + ELA, eslot)

                    bn = (u + GLA) % NBUF
                    en = (u + GLA) % NEBUF

                    @pl.when(jnp.logical_and(k + GLA < n_chunks,
                                             k + GLA >= NBUF))
                    def _(bn=bn):
                        pltpu.make_async_copy(rows_v.at[bn],
                                              acc.at[ebuf.at[0, 1]],
                                              ssem.at[bn]).wait()

                    @pl.when(k + GLA < n_chunks)
                    def _(bn=bn, en=en):
                        wait_edata(en)
                        pltpu.async_copy(yh.at[ebuf.at[en, 0]],
                                         rows_v.at[bn], gsem.at[bn])

                    pltpu.make_async_copy(yh.at[ebuf.at[u, 0]], rows_v.at[b],
                                          gsem.at[b]).wait()

                    def mul_body(gi, carry2, b=b, u=u):
                        wi = ebuf[u, 2, pl.ds(gi * LANES, LANES)]
                        ew16 = lax.bitcast_convert_type(wi, jnp.float32)
                        for e in range(LANES):
                            w = lax.gather(
                                ew16,
                                jnp.full((LANES, 1), e, jnp.int32),
                                lax.GatherDimensionNumbers(
                                    offset_dims=(),
                                    collapsed_slice_dims=(0,),
                                    start_index_map=(0,)),
                                slice_sizes=(1,),
                                mode=lax.GatherScatterMode.PROMISE_IN_BOUNDS)
                            i = gi * LANES + e
                            for j in range(half // LANES):
                                sl = pl.ds(j * LANES, LANES)
                                rows_v[b, i, sl] = rows_v[b, i, sl] * w
                        return carry2

                    lax.fori_loop(0, E_CHUNK // LANES, mul_body, 0)
                    pltpu.async_copy(rows_v.at[b], acc.at[ebuf.at[u, 1]],
                                     ssem.at[b], add=True)
                return carry

            lax.fori_loop(0, n_groups, group_body, 0)
            for b in range(NBUF):
                pltpu.make_async_copy(rows_v.at[b], acc.at[ebuf.at[0, 1]],
                                      ssem.at[b]).wait()

        @pl.when(c == 0)
        def _():
            edge_pass(y_lo)

        @pl.when(c == 1)
        def _():
            edge_pass(y_hi)

        plsc.subcore_barrier()

        for core in range(2):
            col = core * half

            @pl.when(jnp.logical_and(c == core, s < N_TILES - 1))
            def _(col=col):
                pltpu.sync_copy(
                    acc.at[pl.ds(row_base, rows_main)],
                    out.at[pl.ds(row_base, rows_main), pl.ds(col, half)])

            @pl.when(jnp.logical_and(c == core, s == N_TILES - 1))
            def _(col=col):
                pltpu.sync_copy(
                    acc.at[pl.ds(last_base, rows_last)],
                    out.at[pl.ds(last_base, rows_last), pl.ds(col, half)])

    return sc_agg


def _mm_body(x_ref, w_ref, o_ref, *, half):
    x = x_ref[...]
    w = w_ref[...]
    o_ref[0] = jnp.dot(x, w[:, :half], preferred_element_type=jnp.float32)
    o_ref[1] = jnp.dot(x, w[:, half:], preferred_element_type=jnp.float32)


def kernel(x, edge_index, edge_weight, W, b):
    n_nodes, d_in = x.shape
    d_out = W.shape[1]
    n_edges = edge_weight.shape[0]
    half = d_out // 2

    cpt = -(-(n_edges // N_TILES) // (E_CHUNK * NEBUF)) * NEBUF
    n_pad = N_TILES * cpt * E_CHUNK - n_edges
    pad_idx = (jnp.arange(n_pad, dtype=jnp.int32) * 8) % n_nodes
    src = jnp.concatenate([edge_index[0].astype(jnp.int32), pad_idx])
    dst = jnp.concatenate([edge_index[1].astype(jnp.int32), pad_idx])
    ew = jnp.concatenate([edge_weight.astype(jnp.float32),
                          jnp.zeros((n_pad,), jnp.float32)])
    src3 = src.reshape(N_TILES, cpt, E_CHUNK)
    dst3 = dst.reshape(N_TILES, cpt, E_CHUNK)
    ewi3 = lax.bitcast_convert_type(ew, jnp.int32).reshape(
        N_TILES, cpt, E_CHUNK)

    rows_main = (n_nodes // (N_TILES * 8)) * 8
    rows_last = n_nodes - (N_TILES - 1) * rows_main
    bias2 = jnp.broadcast_to(b, (rows_last, d_out))
    bias2 = bias2.reshape(rows_last, 2, half).transpose(1, 0, 2)

    rows_blk = 1000
    grid = (n_nodes // rows_blk,)
    y2 = pl.pallas_call(
        functools.partial(_mm_body, half=half),
        grid=grid,
        in_specs=[
            pl.BlockSpec((rows_blk, d_in), lambda i: (i, 0)),
            pl.BlockSpec((d_in, d_out), lambda i: (0, 0)),
        ],
        out_specs=pl.BlockSpec((2, rows_blk, half), lambda i: (0, i, 0)),
        out_shape=jax.ShapeDtypeStruct((2, n_nodes, half), jnp.float32),
    )(x, W)

    sc_agg = _make_sc_agg(n_nodes, cpt, half)
    return sc_agg(y2[0], y2[1], src3, dst3, ewi3, bias2)


# speedup vs baseline: 1.0277x; 1.0076x over previous
import functools

import jax
import jax.numpy as jnp
from jax import lax
from jax.experimental import pallas as pl
from jax.experimental.pallas import tpu as pltpu
from jax.experimental.pallas import tpu_sc as plsc

LANES = 16
N_TILES = 16
E_CHUNK = 80
NBUF = 4
NEBUF = 8
GLA = 2
ELA = 3


def _make_sc_agg(n_nodes, n_chunks, half):
    rows_main = (n_nodes // (N_TILES * 8)) * 8
    rows_last = n_nodes - (N_TILES - 1) * rows_main
    n_groups = n_chunks // NEBUF
    mesh = plsc.VectorSubcoreMesh(core_axis_name="c", subcore_axis_name="s")

    @functools.partial(
        pl.kernel,
        mesh=mesh,
        out_type=jax.ShapeDtypeStruct((n_nodes, 2 * half), jnp.float32),
        scratch_types=[
            pltpu.VMEM((NEBUF, 3, E_CHUNK), jnp.int32),
            pltpu.VMEM((NBUF, E_CHUNK, half), jnp.float32),
            pltpu.VMEM_SHARED((n_nodes, half), jnp.float32),
            pltpu.SemaphoreType.DMA((NEBUF,)),
            pltpu.SemaphoreType.DMA((NBUF,)),
            pltpu.SemaphoreType.DMA((NBUF,)),
        ],
    )
    def sc_agg(y_lo, y_hi, src3_h, dst3_h, ewi3_h, bias_h, out,
               ebuf, rows_v, acc, isem, gsem, ssem):
        c = lax.axis_index("c")
        s = lax.axis_index("s")
        row_base = pl.multiple_of(s * rows_main, 8)
        last_base = (N_TILES - 1) * rows_main

        @pl.when(s < N_TILES - 1)
        def _():
            pltpu.sync_copy(bias_h.at[c, pl.ds(0, rows_main)],
                            acc.at[pl.ds(row_base, rows_main)])

        @pl.when(s == N_TILES - 1)
        def _():
            pltpu.sync_copy(bias_h.at[c],
                            acc.at[pl.ds(last_base, rows_last)])

        def stage_edata(kk, slot):
            pltpu.async_copy(src3_h.at[s, kk], ebuf.at[slot, 0],
                             isem.at[slot])
            pltpu.async_copy(dst3_h.at[s, kk], ebuf.at[slot, 1],
                             isem.at[slot])
            pltpu.async_copy(ewi3_h.at[s, kk], ebuf.at[slot, 2],
                             isem.at[slot])

        def wait_edata(slot):
            for j in range(3):
                pltpu.make_async_copy(src3_h.at[s, 0], ebuf.at[slot, j],
                                      isem.at[slot]).wait()

        def edge_pass(yh):
            for j in range(ELA):
                stage_edata(j, j)
            for j in range(GLA):
                wait_edata(j)
                pltpu.async_copy(yh.at[ebuf.at[j, 0]], rows_v.at[j],
                                 gsem.at[j])
            plsc.subcore_barrier()

            def group_body(g, carry):
                for u in range(NEBUF):
                    k = g * NEBUF + u
                    b = u % NBUF
                    eslot = (u + ELA) % NEBUF

                    @pl.when(k + ELA < n_chunks)
                    def _(eslot=eslot, k=k):
                        stage_edata(k + ELA, eslot)

                    bn = (u + GLA) % NBUF
                    en = (u + GLA) % NEBUF

                    @pl.when(jnp.logical_and(k + GLA < n_chunks,
                                             k + GLA >= NBUF))
                    def _(bn=bn):
                        pltpu.make_async_copy(rows_v.at[bn],
                                              acc.at[ebuf.at[0, 1]],
                                              ssem.at[bn]).wait()

                    @pl.when(k + GLA < n_chunks)
                    def _(bn=bn, en=en):
                        wait_edata(en)
                        pltpu.async_copy(yh.at[ebuf.at[en, 0]],
                                         rows_v.at[bn], gsem.at[bn])

                    pltpu.make_async_copy(yh.at[ebuf.at[u, 0]], rows_v.at[b],
                                          gsem.at[b]).wait()

                    def mul_body(gi, carry2, b=b, u=u):
                        wi = ebuf[u, 2, pl.ds(gi * LANES, LANES)]
                        ew16 = lax.bitcast_convert_type(wi, jnp.float32)
                        for e in range(LANES):
                            w = lax.gather(
                                ew16,
                                jnp.full((LANES, 1), e, jnp.int32),
                                lax.GatherDimensionNumbers(
                                    offset_dims=(),
                                    collapsed_slice_dims=(0,),
                                    start_index_map=(0,)),
                                slice_sizes=(1,),
                                mode=lax.GatherScatterMode.PROMISE_IN_BOUNDS)
                            i = gi * LANES + e
                            for j in range(half // LANES):
                                sl = pl.ds(j * LANES, LANES)
                                rows_v[b, i, sl] = rows_v[b, i, sl] * w
                        return carry2

                    lax.fori_loop(0, E_CHUNK // LANES, mul_body, 0)
                    pltpu.async_copy(rows_v.at[b], acc.at[ebuf.at[u, 1]],
                                     ssem.at[b], add=True)
                return carry

            lax.fori_loop(0, n_groups, group_body, 0)
            for b in range(NBUF):
                pltpu.make_async_copy(rows_v.at[b], acc.at[ebuf.at[0, 1]],
                                      ssem.at[b]).wait()

        @pl.when(c == 0)
        def _():
            edge_pass(y_lo)

        @pl.when(c == 1)
        def _():
            edge_pass(y_hi)

        plsc.subcore_barrier()

        for core in range(2):
            col = core * half

            @pl.when(jnp.logical_and(c == core, s < N_TILES - 1))
            def _(col=col):
                pltpu.sync_copy(
                    acc.at[pl.ds(row_base, rows_main)],
                    out.at[pl.ds(row_base, rows_main), pl.ds(col, half)])

            @pl.when(jnp.logical_and(c == core, s == N_TILES - 1))
            def _(col=col):
                pltpu.sync_copy(
                    acc.at[pl.ds(last_base, rows_last)],
                    out.at[pl.ds(last_base, rows_last), pl.ds(col, half)])

    return sc_agg


def _mm_body(x_ref, w_ref, o_ref, *, half):
    x = x_ref[...]
    w = w_ref[...]
    o_ref[0] = jnp.dot(x, w[:, :half], preferred_element_type=jnp.float32)
    o_ref[1] = jnp.dot(x, w[:, half:], preferred_element_type=jnp.float32)


def kernel(x, edge_index, edge_weight, W, b):
    n_nodes, d_in = x.shape
    d_out = W.shape[1]
    n_edges = edge_weight.shape[0]
    half = d_out // 2

    cpt = -(-(n_edges // N_TILES) // (E_CHUNK * NEBUF)) * NEBUF
    n_pad = N_TILES * cpt * E_CHUNK - n_edges
    pad_idx = (jnp.arange(n_pad, dtype=jnp.int32) * 8) % n_nodes
    src = jnp.concatenate([edge_index[0].astype(jnp.int32), pad_idx])
    dst = jnp.concatenate([edge_index[1].astype(jnp.int32), pad_idx])
    ew = jnp.concatenate([edge_weight.astype(jnp.float32),
                          jnp.zeros((n_pad,), jnp.float32)])
    src3 = src.reshape(N_TILES, cpt, E_CHUNK)
    dst3 = dst.reshape(N_TILES, cpt, E_CHUNK)
    ewi3 = lax.bitcast_convert_type(ew, jnp.int32).reshape(
        N_TILES, cpt, E_CHUNK)

    rows_main = (n_nodes // (N_TILES * 8)) * 8
    rows_last = n_nodes - (N_TILES - 1) * rows_main
    bias2 = jnp.broadcast_to(b, (rows_last, d_out))
    bias2 = bias2.reshape(rows_last, 2, half).transpose(1, 0, 2)

    rows_blk = 1000
    grid = (n_nodes // rows_blk,)
    y2 = pl.pallas_call(
        functools.partial(_mm_body, half=half),
        grid=grid,
        in_specs=[
            pl.BlockSpec((rows_blk, d_in), lambda i: (i, 0)),
            pl.BlockSpec((d_in, d_out), lambda i: (0, 0)),
        ],
        out_specs=pl.BlockSpec((2, rows_blk, half), lambda i: (0, i, 0)),
        out_shape=jax.ShapeDtypeStruct((2, n_nodes, half), jnp.float32),
    )(x, W)

    sc_agg = _make_sc_agg(n_nodes, cpt, half)
    return sc_agg(y2[0], y2[1], src3, dst3, ewi3, bias2)
